# baseline (speedup 1.0000x reference)
import functools
import numpy as np
import jax
import jax.numpy as jnp
from jax import lax
from jax.experimental import pallas as pl
from jax.experimental.pallas import tpu as pltpu
from jax.experimental.pallas import tpu_sc as plsc


def _enc_table(S, E):
    pos = np.arange(S, dtype=np.float64)[:, None]
    i = np.arange(E, dtype=np.float64)[None, :]
    angle = pos / np.power(10000.0, (i - np.mod(i, 2)) / E)
    enc = np.array(angle)
    enc[:, 0::2] = np.sin(angle[:, 0::2])
    enc[:, 1::2] = np.cos(angle[:, 1::2])
    return jnp.asarray(enc, dtype=jnp.float32)


_NC, _NS = 2, 16
_NW = _NC * _NS
_RB = 32
_BSC = 512


def _sc_broadcast(enc, B, S, E):
    CHUNK = S // _NW
    NT = CHUNK // _RB

    mesh = plsc.VectorSubcoreMesh(core_axis_name="c", subcore_axis_name="s")

    @functools.partial(
        pl.kernel,
        out_type=jax.ShapeDtypeStruct((B, S, E), jnp.float32),
        mesh=mesh,
        scratch_types=[
            pltpu.VMEM((2, _RB, E), jnp.float32),
            pltpu.SemaphoreType.DMA,
            pltpu.SemaphoreType.DMA,
            pltpu.SemaphoreType.DMA,
            pltpu.SemaphoreType.DMA,
        ],
        compiler_params=pltpu.CompilerParams(
            use_tc_tiling_on_sc=True, needs_layout_passes=False),
    )
    def bcast_kernel(enc_hbm, out_hbm, enc_v, ld0, ld1, st0, st1):
        wid = lax.axis_index("s") * _NC + lax.axis_index("c")
        base = wid * CHUNK
        ld = (ld0, ld1)
        st = (st0, st1)

        def start_load(t):
            p = t % 2
            r0 = base + t * _RB
            pltpu.async_copy(enc_hbm.at[pl.ds(r0, _RB), :], enc_v.at[p], ld[p])

        def wait_load(t):
            p = t % 2
            r0 = base + t * _RB
            pltpu.make_async_copy(
                enc_hbm.at[pl.ds(r0, _RB), :], enc_v.at[p], ld[p]).wait()

        def start_store(t):
            p = t % 2
            r0 = base + t * _RB
            for b in range(B):
                pltpu.async_copy(
                    enc_v.at[p], out_hbm.at[b, pl.ds(r0, _RB), :], st[p])

        def wait_store(t):
            p = t % 2
            r0 = base + t * _RB
            for b in range(B):
                pltpu.make_async_copy(
                    enc_v.at[p], out_hbm.at[b, pl.ds(r0, _RB), :],
                    st[p]).wait()

        start_load(0)
        for t in range(NT):
            if t + 1 < NT:
                if t >= 1:
                    wait_store(t - 1)
                start_load(t + 1)
            wait_load(t)
            start_store(t)
        wait_store(NT - 2)
        wait_store(NT - 1)

    return bcast_kernel(enc)


def _tc_scan(x, B, S, E):
    NK = S // _BSC

    def body(x_ref, f_ref):
        b = pl.program_id(0)
        k = pl.program_id(1)
        f_ref[b, k] = jnp.sum((x_ref[...] == 0.0).astype(jnp.float32))

    return pl.pallas_call(
        body,
        grid=(B, NK),
        in_specs=[pl.BlockSpec((1, _BSC, E), lambda b, k: (b, k, 0))],
        out_specs=pl.BlockSpec(
            (B, NK), lambda b, k: (0, 0), memory_space=pltpu.SMEM),
        out_shape=jax.ShapeDtypeStruct((B, NK), jnp.float32),
    )(x)


def _tc_patch(flags, x, enc, out0, B, S, E):
    NK = S // _BSC

    def body(fl_vec_ref, fl_ref, x_ref, enc_ref, out0_ref, o_ref,
             xb_ref, eb_ref, sem):
        glob = jnp.max(fl_vec_ref[...])

        @pl.when(glob > 0.0)
        def _():
            def b_loop(b, carry):
                def k_loop(k, carry2):
                    f = fl_ref[b, k]

                    @pl.when(f > 0.0)
                    def _patch():
                        r0 = k * _BSC
                        pltpu.make_async_copy(
                            enc_ref.at[pl.ds(r0, _BSC), :], eb_ref,
                            sem).start()
                        pltpu.make_async_copy(
                            enc_ref.at[pl.ds(r0, _BSC), :], eb_ref,
                            sem).wait()
                        pltpu.make_async_copy(
                            x_ref.at[b, pl.ds(r0, _BSC), :], xb_ref,
                            sem).start()
                        pltpu.make_async_copy(
                            x_ref.at[b, pl.ds(r0, _BSC), :], xb_ref,
                            sem).wait()
                        xv = xb_ref[...]
                        xb_ref[...] = jnp.where(xv == 0.0, xv, eb_ref[...])
                        pltpu.make_async_copy(
                            xb_ref, o_ref.at[b, pl.ds(r0, _BSC), :],
                            sem).start()
                        pltpu.make_async_copy(
                            xb_ref, o_ref.at[b, pl.ds(r0, _BSC), :],
                            sem).wait()

                    return carry2

                return lax.fori_loop(0, NK, k_loop, carry)

            lax.fori_loop(0, B, b_loop, 0)

    return pl.pallas_call(
        body,
        in_specs=[
            pl.BlockSpec(memory_space=pltpu.VMEM),
            pl.BlockSpec(memory_space=pltpu.SMEM),
            pl.BlockSpec(memory_space=pl.ANY),
            pl.BlockSpec(memory_space=pl.ANY),
            pl.BlockSpec(memory_space=pl.ANY),
        ],
        out_specs=pl.BlockSpec(memory_space=pl.ANY),
        out_shape=jax.ShapeDtypeStruct((B, S, E), jnp.float32),
        scratch_shapes=[
            pltpu.VMEM((_BSC, E), jnp.float32),
            pltpu.VMEM((_BSC, E), jnp.float32),
            pltpu.SemaphoreType.DMA,
        ],
        input_output_aliases={4: 0},
    )(flags, flags, x, enc, out0)


@functools.partial(jax.jit, static_argnums=(2, 3, 4))
def _run(x, enc, B, S, E):
    out0 = _sc_broadcast(enc, B, S, E)
    flags = _tc_scan(x, B, S, E)
    return _tc_patch(flags, x, enc, out0, B, S, E)


def kernel(x):
    B, S, E = x.shape
    enc = _enc_table(S, E)
    return _run(x, enc, B, S, E)


# speedup vs baseline: 1.0038x; 1.0038x over previous
import functools
import numpy as np
import jax
import jax.numpy as jnp
from jax import lax
from jax.experimental import pallas as pl
from jax.experimental.pallas import tpu as pltpu
from jax.experimental.pallas import tpu_sc as plsc


def _enc_table(S, E):
    pos = np.arange(S, dtype=np.float64)[:, None]
    i = np.arange(E, dtype=np.float64)[None, :]
    angle = pos / np.power(10000.0, (i - np.mod(i, 2)) / E)
    enc = np.array(angle)
    enc[:, 0::2] = np.sin(angle[:, 0::2])
    enc[:, 1::2] = np.cos(angle[:, 1::2])
    return jnp.asarray(enc, dtype=jnp.float32)


_NC, _NS = 2, 16
_NW = _NC * _NS
_RB = 32
_BSC = 512


def _sc_broadcast(enc3, B, S, E):
    CHUNK = S // _NW
    NT = CHUNK // _RB

    mesh = plsc.VectorSubcoreMesh(core_axis_name="c", subcore_axis_name="s")

    @functools.partial(
        pl.kernel,
        out_type=jax.ShapeDtypeStruct((B, S, E), jnp.float32),
        mesh=mesh,
        scratch_types=[
            pltpu.VMEM((2, _RB, E), jnp.float32),
            pltpu.SemaphoreType.DMA,
            pltpu.SemaphoreType.DMA,
            pltpu.SemaphoreType.DMA,
            pltpu.SemaphoreType.DMA,
        ],
        compiler_params=pltpu.CompilerParams(
            use_tc_tiling_on_sc=True, needs_layout_passes=False),
    )
    def bcast_kernel(enc_hbm, out_hbm, enc_v, ld0, ld1, st0, st1):
        wid = lax.axis_index("s") * _NC + lax.axis_index("c")
        base = wid * CHUNK
        ld = (ld0, ld1)
        st = (st0, st1)

        def start_load(t):
            p = t % 2
            r0 = base + t * _RB
            pltpu.async_copy(
                enc_hbm.at[0, pl.ds(r0, _RB), :], enc_v.at[p], ld[p])

        def wait_load(t):
            p = t % 2
            r0 = base + t * _RB
            pltpu.make_async_copy(
                enc_hbm.at[0, pl.ds(r0, _RB), :], enc_v.at[p], ld[p]).wait()

        def start_store(t):
            p = t % 2
            r0 = base + t * _RB
            for b in range(B):
                pltpu.async_copy(
                    enc_v.at[p], out_hbm.at[b, pl.ds(r0, _RB), :], st[p])

        def wait_store(t):
            p = t % 2
            r0 = base + t * _RB
            for b in range(B):
                pltpu.make_async_copy(
                    enc_v.at[p], out_hbm.at[b, pl.ds(r0, _RB), :],
                    st[p]).wait()

        start_load(0)
        for t in range(NT):
            if t + 1 < NT:
                if t >= 1:
                    wait_store(t - 1)
                start_load(t + 1)
            wait_load(t)
            start_store(t)
        wait_store(NT - 2)
        wait_store(NT - 1)

    return bcast_kernel(enc3)


def _tc_scan(x, B, S, E):
    NK = S // _BSC

    def body(x_ref, f_ref):
        b = pl.program_id(0)
        k = pl.program_id(1)
        f_ref[b, k] = jnp.where(jnp.min(jnp.abs(x_ref[...])) == 0.0, 1.0, 0.0)

    return pl.pallas_call(
        body,
        grid=(B, NK),
        in_specs=[pl.BlockSpec((1, _BSC, E), lambda b, k: (b, k, 0))],
        out_specs=pl.BlockSpec(
            (B, NK), lambda b, k: (0, 0), memory_space=pltpu.SMEM),
        out_shape=jax.ShapeDtypeStruct((B, NK), jnp.float32),
    )(x)


def _tc_patch(flags, x, enc, out0, B, S, E):
    NK = S // _BSC

    def body(fl_vec_ref, fl_ref, x_ref, enc_ref, out0_ref, o_ref,
             xb_ref, eb_ref, sem):
        glob = jnp.max(fl_vec_ref[...])

        @pl.when(glob > 0.0)
        def _():
            def b_loop(b, carry):
                def k_loop(k, carry2):
                    f = fl_ref[b, k]

                    @pl.when(f > 0.0)
                    def _patch():
                        r0 = k * _BSC
                        pltpu.make_async_copy(
                            enc_ref.at[pl.ds(r0, _BSC), :], eb_ref,
                            sem).start()
                        pltpu.make_async_copy(
                            enc_ref.at[pl.ds(r0, _BSC), :], eb_ref,
                            sem).wait()
                        pltpu.make_async_copy(
                            x_ref.at[b, pl.ds(r0, _BSC), :], xb_ref,
                            sem).start()
                        pltpu.make_async_copy(
                            x_ref.at[b, pl.ds(r0, _BSC), :], xb_ref,
                            sem).wait()
                        xv = xb_ref[...]
                        xb_ref[...] = jnp.where(xv == 0.0, xv, eb_ref[...])
                        pltpu.make_async_copy(
                            xb_ref, o_ref.at[b, pl.ds(r0, _BSC), :],
                            sem).start()
                        pltpu.make_async_copy(
                            xb_ref, o_ref.at[b, pl.ds(r0, _BSC), :],
                            sem).wait()

                    return carry2

                return lax.fori_loop(0, NK, k_loop, carry)

            lax.fori_loop(0, B, b_loop, 0)

    return pl.pallas_call(
        body,
        in_specs=[
            pl.BlockSpec(memory_space=pltpu.VMEM),
            pl.BlockSpec(memory_space=pltpu.SMEM),
            pl.BlockSpec(memory_space=pl.ANY),
            pl.BlockSpec(memory_space=pl.ANY),
            pl.BlockSpec(memory_space=pl.ANY),
        ],
        out_specs=pl.BlockSpec(memory_space=pl.ANY),
        out_shape=jax.ShapeDtypeStruct((B, S, E), jnp.float32),
        scratch_shapes=[
            pltpu.VMEM((_BSC, E), jnp.float32),
            pltpu.VMEM((_BSC, E), jnp.float32),
            pltpu.SemaphoreType.DMA,
        ],
        input_output_aliases={4: 0},
    )(flags, flags, x, enc, out0)


@functools.partial(jax.jit, static_argnums=(2, 3, 4))
def _run(x, enc, B, S, E):
    out0 = _sc_broadcast(enc.reshape(1, S, E), B, S, E)
    flags = _tc_scan(x, B, S, E)
    return _tc_patch(flags, x, enc, out0, B, S, E)


def kernel(x):
    B, S, E = x.shape
    enc = _enc_table(S, E)
    return _run(x, enc, B, S, E)


# speedup vs baseline: 1.1707x; 1.1662x over previous
import functools
import numpy as np
import jax
import jax.numpy as jnp
from jax import lax
from jax.experimental import pallas as pl
from jax.experimental.pallas import tpu as pltpu
from jax.experimental.pallas import tpu_sc as plsc


def _enc_table(S, E):
    pos = np.arange(S, dtype=np.float64)[:, None]
    i = np.arange(E, dtype=np.float64)[None, :]
    angle = pos / np.power(10000.0, (i - np.mod(i, 2)) / E)
    enc = np.array(angle)
    enc[:, 0::2] = np.sin(angle[:, 0::2])
    enc[:, 1::2] = np.cos(angle[:, 1::2])
    return jnp.asarray(enc, dtype=jnp.float32)


_NC, _NS = 2, 16
_NW = _NC * _NS
_RT = 8
_BSC = 512
_LANE = 16


def _sc_scan(x, B, S, E, RSC):
    C = RSC // _NW
    NTW = C // _RT

    mesh = plsc.VectorSubcoreMesh(core_axis_name="c", subcore_axis_name="s")

    @functools.partial(
        pl.kernel,
        out_type=jax.ShapeDtypeStruct((_NW, _LANE), jnp.float32),
        mesh=mesh,
        scratch_types=[
            pltpu.VMEM((2, B * _RT, E), jnp.float32),
            pltpu.VMEM((_LANE,), jnp.float32),
            pltpu.SemaphoreType.DMA,
            pltpu.SemaphoreType.DMA,
            pltpu.SemaphoreType.DMA,
        ],
        compiler_params=pltpu.CompilerParams(
            use_tc_tiling_on_sc=True, needs_layout_passes=False),
    )
    def scan_kernel(x_hbm, fl_hbm, xv, flv, ld0, ld1, stsem):
        wid = lax.axis_index("s") * _NC + lax.axis_index("c")
        base = wid * C
        ld = (ld0, ld1)
        lane = lax.iota(jnp.int32, _LANE)

        def start_load(p, t):
            r0 = base + t * _RT
            for b in range(B):
                pltpu.async_copy(
                    x_hbm.at[b, pl.ds(r0, _RT), :],
                    xv.at[p, pl.ds(b * _RT, _RT), :], ld[p])

        def wait_load(p, t):
            r0 = base + t * _RT
            for b in range(B):
                pltpu.make_async_copy(
                    x_hbm.at[b, pl.ds(r0, _RT), :],
                    xv.at[p, pl.ds(b * _RT, _RT), :], ld[p]).wait()

        def compute(p, t, fl):
            acc = jnp.full((_LANE,), 1.0, jnp.float32)
            for rr in range(B * _RT):
                @plsc.parallel_loop(0, E, _LANE, unroll=4, carry=acc)
                def acc_loop(j, a):
                    return jnp.minimum(a, jnp.abs(xv[p, rr, pl.ds(j, _LANE)]))
                acc = acc_loop
            nz = plsc.all_reduce_population_count(acc == 0.0)
            return jnp.where((lane == t) & (nz > 0), 1.0, fl)

        fl0 = jnp.zeros((_LANE,), jnp.float32)
        start_load(0, 0)
        start_load(1, 1)

        def body(g, fl):
            t0 = 2 * g
            wait_load(0, t0)
            fl = compute(0, t0, fl)
            start_load(0, t0 + 2)
            wait_load(1, t0 + 1)
            fl = compute(1, t0 + 1, fl)
            start_load(1, t0 + 3)
            return fl

        fl0 = lax.fori_loop(0, (NTW - 2) // 2, body, fl0)
        wait_load(0, NTW - 2)
        fl0 = compute(0, NTW - 2, fl0)
        wait_load(1, NTW - 1)
        fl0 = compute(1, NTW - 1, fl0)

        flv[...] = fl0
        pltpu.async_copy(flv, fl_hbm.at[wid], stsem)
        pltpu.make_async_copy(flv, fl_hbm.at[wid], stsem).wait()

    return scan_kernel(x)


def _tc_main(x, enc, B, S, E, KS):
    NK = S // _BSC

    def body(x_ref, enc_ref, o_ref):
        k = pl.program_id(0)

        @pl.when(k < KS)
        def _bcast():
            o_ref[...] = enc_ref[...][None, :, :]

        @pl.when(k >= KS)
        def _select():
            xv = x_ref[...]
            o_ref[...] = jnp.where(xv == 0.0, xv, enc_ref[...][None, :, :])

    return pl.pallas_call(
        body,
        grid=(NK, B),
        in_specs=[
            pl.BlockSpec(
                (1, _BSC, E),
                lambda k, b: (jnp.where(k < KS, 0, b),
                              jnp.maximum(k, KS), 0)),
            pl.BlockSpec((_BSC, E), lambda k, b: (k, 0)),
        ],
        out_specs=pl.BlockSpec((1, _BSC, E), lambda k, b: (b, k, 0)),
        out_shape=jax.ShapeDtypeStruct((B, S, E), jnp.float32),
    )(x, enc)


def _tc_patch(flags, x, enc, out0, B, S, E, RSC):
    C = RSC // _NW
    NTW = C // _RT

    def body(fl_vec_ref, fl_ref, x_ref, enc_ref, out0_ref, o_ref,
             xb_ref, eb_ref, sem):
        glob = jnp.max(fl_vec_ref[...])

        @pl.when(glob > 0.0)
        def _():
            def w_loop(w, carry):
                def t_loop(t, carry2):
                    f = fl_ref[w, t]

                    @pl.when(f > 0.0)
                    def _patch():
                        r0 = w * C + t * _RT
                        pltpu.make_async_copy(
                            enc_ref.at[pl.ds(r0, _RT), :], eb_ref,
                            sem).start()
                        pltpu.make_async_copy(
                            enc_ref.at[pl.ds(r0, _RT), :], eb_ref,
                            sem).wait()
                        for b in range(B):
                            pltpu.make_async_copy(
                                x_ref.at[b, pl.ds(r0, _RT), :], xb_ref,
                                sem).start()
                            pltpu.make_async_copy(
                                x_ref.at[b, pl.ds(r0, _RT), :], xb_ref,
                                sem).wait()
                            xv = xb_ref[...]
                            xb_ref[...] = jnp.where(
                                xv == 0.0, xv, eb_ref[...])
                            pltpu.make_async_copy(
                                xb_ref, o_ref.at[b, pl.ds(r0, _RT), :],
                                sem).start()
                            pltpu.make_async_copy(
                                xb_ref, o_ref.at[b, pl.ds(r0, _RT), :],
                                sem).wait()

                    return carry2

                return lax.fori_loop(0, NTW, t_loop, carry)

            lax.fori_loop(0, _NW, w_loop, 0)

    return pl.pallas_call(
        body,
        in_specs=[
            pl.BlockSpec(memory_space=pltpu.VMEM),
            pl.BlockSpec(memory_space=pltpu.SMEM),
            pl.BlockSpec(memory_space=pl.ANY),
            pl.BlockSpec(memory_space=pl.ANY),
            pl.BlockSpec(memory_space=pl.ANY),
        ],
        out_specs=pl.BlockSpec(memory_space=pl.ANY),
        out_shape=jax.ShapeDtypeStruct((B, S, E), jnp.float32),
        scratch_shapes=[
            pltpu.VMEM((_RT, E), jnp.float32),
            pltpu.VMEM((_RT, E), jnp.float32),
            pltpu.SemaphoreType.DMA,
        ],
        input_output_aliases={4: 0},
    )(flags, flags, x, enc, out0)


@functools.partial(jax.jit, static_argnums=(2, 3, 4))
def _run(x, enc, B, S, E):
    KS = S // _BSC - 1
    RSC = KS * _BSC
    flags = _sc_scan(x, B, S, E, RSC)
    out0 = _tc_main(x, enc, B, S, E, KS)
    return _tc_patch(flags, x, enc, out0, B, S, E, RSC)


def kernel(x):
    B, S, E = x.shape
    enc = _enc_table(S, E)
    return _run(x, enc, B, S, E)


# speedup vs baseline: 1.2044x; 1.0288x over previous
import functools
import numpy as np
import jax
import jax.numpy as jnp
from jax import lax
from jax.experimental import pallas as pl
from jax.experimental.pallas import tpu as pltpu
from jax.experimental.pallas import tpu_sc as plsc


def _enc_table(S, E):
    pos = np.arange(S, dtype=np.float64)[:, None]
    i = np.arange(E, dtype=np.float64)[None, :]
    angle = pos / np.power(10000.0, (i - np.mod(i, 2)) / E)
    enc = np.array(angle)
    enc[:, 0::2] = np.sin(angle[:, 0::2])
    enc[:, 1::2] = np.cos(angle[:, 1::2])
    return jnp.asarray(enc, dtype=jnp.float32)


_NC, _NS = 2, 16
_NW = _NC * _NS
_RT = 8
_BSC = 512
_LANE = 16


def _sc_scan(x, B, S, E, RSC):
    C = RSC // _NW
    NTW = C // _RT

    mesh = plsc.VectorSubcoreMesh(core_axis_name="c", subcore_axis_name="s")

    @functools.partial(
        pl.kernel,
        out_type=jax.ShapeDtypeStruct((_NW, _LANE), jnp.float32),
        mesh=mesh,
        scratch_types=[
            pltpu.VMEM((2, B * _RT, E), jnp.float32),
            pltpu.VMEM((_LANE,), jnp.float32),
            pltpu.SemaphoreType.DMA,
            pltpu.SemaphoreType.DMA,
            pltpu.SemaphoreType.DMA,
        ],
        compiler_params=pltpu.CompilerParams(
            use_tc_tiling_on_sc=True, needs_layout_passes=False),
    )
    def scan_kernel(x_hbm, fl_hbm, xv, flv, ld0, ld1, stsem):
        wid = lax.axis_index("s") * _NC + lax.axis_index("c")
        base = wid * C
        ld = (ld0, ld1)
        lane = lax.iota(jnp.int32, _LANE)

        def start_load(p, t):
            r0 = base + t * _RT
            for b in range(B):
                pltpu.async_copy(
                    x_hbm.at[b, pl.ds(r0, _RT), :],
                    xv.at[p, pl.ds(b * _RT, _RT), :], ld[p])

        def wait_load(p, t):
            r0 = base + t * _RT
            for b in range(B):
                pltpu.make_async_copy(
                    x_hbm.at[b, pl.ds(r0, _RT), :],
                    xv.at[p, pl.ds(b * _RT, _RT), :], ld[p]).wait()

        def compute(p, t, fl):
            acc = jnp.full((_LANE,), 1.0, jnp.float32)
            for rr in range(B * _RT):
                @plsc.parallel_loop(0, E, _LANE, unroll=4, carry=acc)
                def acc_loop(j, a):
                    return jnp.minimum(a, jnp.abs(xv[p, rr, pl.ds(j, _LANE)]))
                acc = acc_loop
            nz = plsc.all_reduce_population_count(acc == 0.0)
            return jnp.where((lane == t) & (nz > 0), 1.0, fl)

        fl0 = jnp.zeros((_LANE,), jnp.float32)
        start_load(0, 0)
        start_load(1, 1)

        def body(g, fl):
            t0 = 2 * g
            wait_load(0, t0)
            fl = compute(0, t0, fl)
            start_load(0, t0 + 2)
            wait_load(1, t0 + 1)
            fl = compute(1, t0 + 1, fl)
            start_load(1, t0 + 3)
            return fl

        fl0 = lax.fori_loop(0, (NTW - 2) // 2, body, fl0)
        wait_load(0, NTW - 2)
        fl0 = compute(0, NTW - 2, fl0)
        wait_load(1, NTW - 1)
        fl0 = compute(1, NTW - 1, fl0)

        flv[...] = fl0
        pltpu.async_copy(flv, fl_hbm.at[wid], stsem)
        pltpu.make_async_copy(flv, fl_hbm.at[wid], stsem).wait()

    return scan_kernel(x)


def _tc_main(x, enc, B, S, E, KS):
    NK = S // _BSC

    def body(x_ref, enc_ref, o_ref):
        k = pl.program_id(0)

        @pl.when(k < KS)
        def _bcast():
            o_ref[...] = enc_ref[...][None, :, :]

        @pl.when(k >= KS)
        def _select():
            xv = x_ref[...]
            o_ref[...] = jnp.where(xv == 0.0, xv, enc_ref[...][None, :, :])

    return pl.pallas_call(
        body,
        grid=(NK, B),
        in_specs=[
            pl.BlockSpec(
                (1, _BSC, E),
                lambda k, b: (jnp.where(k < KS, 0, b),
                              jnp.maximum(k, KS), 0)),
            pl.BlockSpec((_BSC, E), lambda k, b: (k, 0)),
        ],
        out_specs=pl.BlockSpec((1, _BSC, E), lambda k, b: (b, k, 0)),
        out_shape=jax.ShapeDtypeStruct((B, S, E), jnp.float32),
    )(x, enc)


def _tc_patch(flags, x, enc, out0, B, S, E, RSC):
    C = RSC // _NW
    NTW = C // _RT

    def body(fl_vec_ref, fl_ref, x_ref, enc_ref, out0_ref, o_ref,
             xb_ref, eb_ref, sem):
        glob = jnp.max(fl_vec_ref[...])

        @pl.when(glob > 0.0)
        def _():
            def w_loop(w, carry):
                def t_loop(t, carry2):
                    f = fl_ref[w, t]

                    @pl.when(f > 0.0)
                    def _patch():
                        r0 = w * C + t * _RT
                        pltpu.make_async_copy(
                            enc_ref.at[pl.ds(r0, _RT), :], eb_ref,
                            sem).start()
                        pltpu.make_async_copy(
                            enc_ref.at[pl.ds(r0, _RT), :], eb_ref,
                            sem).wait()
                        for b in range(B):
                            pltpu.make_async_copy(
                                x_ref.at[b, pl.ds(r0, _RT), :], xb_ref,
                                sem).start()
                            pltpu.make_async_copy(
                                x_ref.at[b, pl.ds(r0, _RT), :], xb_ref,
                                sem).wait()
                            xv = xb_ref[...]
                            xb_ref[...] = jnp.where(
                                xv == 0.0, xv, eb_ref[...])
                            pltpu.make_async_copy(
                                xb_ref, o_ref.at[b, pl.ds(r0, _RT), :],
                                sem).start()
                            pltpu.make_async_copy(
                                xb_ref, o_ref.at[b, pl.ds(r0, _RT), :],
                                sem).wait()

                    return carry2

                return lax.fori_loop(0, NTW, t_loop, carry)

            lax.fori_loop(0, _NW, w_loop, 0)

    return pl.pallas_call(
        body,
        in_specs=[
            pl.BlockSpec(memory_space=pltpu.VMEM),
            pl.BlockSpec(memory_space=pltpu.SMEM),
            pl.BlockSpec(memory_space=pl.ANY),
            pl.BlockSpec(memory_space=pl.ANY),
            pl.BlockSpec(memory_space=pl.ANY),
        ],
        out_specs=pl.BlockSpec(memory_space=pl.ANY),
        out_shape=jax.ShapeDtypeStruct((B, S, E), jnp.float32),
        scratch_shapes=[
            pltpu.VMEM((_RT, E), jnp.float32),
            pltpu.VMEM((_RT, E), jnp.float32),
            pltpu.SemaphoreType.DMA,
        ],
        input_output_aliases={4: 0},
    )(flags, flags, x, enc, out0)


@functools.partial(jax.jit, static_argnums=(2, 3, 4))
def _run(x, enc, B, S, E):
    KS = S // _BSC - 2
    RSC = KS * _BSC
    flags = _sc_scan(x, B, S, E, RSC)
    out0 = _tc_main(x, enc, B, S, E, KS)
    return _tc_patch(flags, x, enc, out0, B, S, E, RSC)


def kernel(x):
    B, S, E = x.shape
    enc = _enc_table(S, E)
    return _run(x, enc, B, S, E)
